# baseline (speedup 1.0000x reference)
import functools

import jax
import jax.numpy as jnp
from jax import lax
from jax.experimental import pallas as pl
from jax.experimental.pallas import tpu as pltpu
from jax.experimental.pallas import tpu_sc as plsc

N0, N1, N2 = 10000, 4000, 1000
E1, E2 = 320000, 64000
D = 128
DP = 144
NC, NS = 2, 16
NW = NC * NS
NSEG = 1024
CH = 128
NB = 4


def _sc_aggregate(n_rows, epw):
  mesh = plsc.VectorSubcoreMesh(
      core_axis_name="c", subcore_axis_name="s", num_cores=NC,
      num_subcores=NS)
  zrows = NSEG // NS
  qcap = epw + NB * CH

  @functools.partial(
      pl.kernel,
      out_type=jax.ShapeDtypeStruct((NC, NSEG, DP), jnp.float32),
      mesh=mesh,
      scratch_types=[
          pltpu.VMEM((epw,), jnp.int32),
          pltpu.VMEM((epw,), jnp.int32),
          pltpu.VMEM((qcap,), jnp.int32),
          pltpu.VMEM((qcap,), jnp.int32),
          pltpu.VMEM((NB, CH), jnp.int32),
          pltpu.VMEM((NB, CH), jnp.int32),
          pltpu.VMEM((NB, CH, DP), jnp.float32),
          pltpu.VMEM_SHARED((NSEG, DP), jnp.float32),
          pltpu.SemaphoreType.DMA((NB,)),
          pltpu.SemaphoreType.DMA((NB,)),
      ],
      compiler_params=pltpu.CompilerParams(use_tc_tiling_on_sc=False,
                                           needs_layout_passes=False),
  )
  def agg(table_hbm, edges_hbm, out_hbm, src_v, dst_v, qs, qd, qs2, qd2,
          rows, acc, semg, sems):
    cid = lax.axis_index("c")
    sid = lax.axis_index("s")
    wid = cid * NS + sid

    zero = jnp.zeros((16,), jnp.float32)
    for r in range(zrows):
      for k in range(DP // 16):
        rows[0, r, pl.ds(k * 16, 16)] = zero
    pltpu.sync_copy(rows.at[0, pl.ds(0, zrows)],
                    acc.at[pl.ds(sid * zrows, zrows)])

    iota = lax.iota(jnp.int32, 16)

    pltpu.sync_copy(edges_hbm.at[0, pl.ds(wid * epw, epw)], src_v)
    pltpu.sync_copy(edges_hbm.at[1, pl.ds(wid * epw, epw)], dst_v)

    def scan_body(i, qn):
      off = pl.multiple_of(i * 16, 16)
      s = src_v[pl.ds(off, 16)]
      d = dst_v[pl.ds(off, 16)]
      m = d < NSEG
      pos = qn + plsc.cumsum(m.astype(jnp.int32)) - 1
      plsc.store_scatter(qs, [pos], s, mask=m)
      plsc.store_scatter(qd, [pos], d, mask=m)
      return qn + plsc.all_reduce_population_count(m)

    qn_v = lax.fori_loop(0, epw // 16, scan_body, jnp.zeros((16,), jnp.int32))
    for k in range(((NB - 1) * CH) // 16):
      pos = qn_v + k * 16 + iota
      plsc.store_scatter(qs, [pos], jnp.zeros((16,), jnp.int32))
      plsc.store_scatter(qd, [pos], jnp.full((16,), NSEG - 8, jnp.int32))
    qn = jnp.max(qn_v)
    nch = jnp.maximum(qn // CH + 1, NB - 1)
    plsc.subcore_barrier()

    def stage(c, b):
      for k in range(CH // 16):
        pos = c * CH + k * 16 + iota
        qs2[b, pl.ds(k * 16, 16)] = plsc.load_gather(qs, [pos])
        qd2[b, pl.ds(k * 16, 16)] = plsc.load_gather(qd, [pos])

    def gather(b):
      return pltpu.make_async_copy(
          table_hbm.at[qs2.at[b]], rows.at[b], semg.at[b])

    def scatter_start(b):
      pltpu.async_copy(rows.at[b], acc.at[qd2.at[b]], sems.at[b], add=True)

    def scatter_wait(b):
      pltpu.make_async_copy(rows.at[b], acc.at[qd2.at[b]], sems.at[b]).wait()

    for b in range(NB - 1):
      stage(b, b)
      gather(b).start()

    def body(c, carry):
      b = c % NB
      ob = (c + NB - 1) % NB
      gather(b).wait()
      scatter_start(b)

      @pl.when(c >= 1)
      def _drain():
        scatter_wait(ob)

      stage(c + NB - 1, ob)
      gather(ob).start()
      return carry

    lax.fori_loop(0, nch - (NB - 1), body, 0)

    def tail(k, carry):
      c = nch - (NB - 1) + k
      b = c % NB
      gather(b).wait()
      pltpu.sync_copy(rows.at[b], acc.at[qd2.at[b]], add=True)
      return carry

    lax.fori_loop(0, NB - 1, tail, 0)

    @pl.when(nch > NB - 1)
    def _drain_last_async():
      scatter_wait((nch - NB) % NB)

    plsc.subcore_barrier()
    pltpu.sync_copy(acc.at[pl.ds(sid * zrows, zrows)],
                    out_hbm.at[cid, pl.ds(sid * zrows, zrows)])

  return agg


def _tc_layer1(p, x1, W1l, W1r, b1):
  def body(p_ref, x_ref, wl_ref, wr_ref, b_ref, out_ref):
    ps = p_ref[0, :N2] + p_ref[1, :N2]
    s = ps[:, :D]
    cnt = ps[:, D:D + 1]
    mean = s / jnp.maximum(cnt, 1.0)
    h = mean @ wl_ref[...] + x_ref[...] @ wr_ref[...] + b_ref[...]
    h = jnp.maximum(h, 0.0)
    hp = jnp.concatenate(
        [h, jnp.ones((N2, 1), jnp.float32),
         jnp.zeros((N2, DP - D - 1), jnp.float32)], axis=1)
    out_ref[...] = jnp.concatenate(
        [hp, jnp.zeros((NSEG - N2, DP), jnp.float32)], axis=0)

  return pl.pallas_call(
      body,
      out_shape=jax.ShapeDtypeStruct((NSEG, DP), jnp.float32),
  )(p, x1, W1l, W1r, b1.reshape(1, D))


def _tc_layer2(p, h, W2l, W2r, b2):
  def body(p_ref, h_ref, wl_ref, wr_ref, b_ref, out_ref):
    ps = p_ref[0, :N2] + p_ref[1, :N2]
    s = ps[:, :D]
    cnt = ps[:, D:D + 1]
    mean = s / jnp.maximum(cnt, 1.0)
    z = mean @ wl_ref[...] + h_ref[:N2, :D] @ wr_ref[...] + b_ref[...]
    m = jnp.max(z, axis=-1, keepdims=True)
    e = jnp.exp(z - m)
    lse = jnp.log(jnp.sum(e, axis=-1, keepdims=True)) + m
    out_ref[...] = z - lse

  return pl.pallas_call(
      body,
      out_shape=jax.ShapeDtypeStruct((N2, D), jnp.float32),
  )(p, h, W2l, W2r, b2.reshape(1, D))


def kernel(x, edge_index1, edge_index2, W1l, W1r, b1, W2l, W2r, b2):
  pad = jnp.concatenate(
      [jnp.ones((N1, 1), jnp.float32),
       jnp.zeros((N1, DP - D - 1), jnp.float32)], axis=1)
  xpad = jnp.concatenate([x[:N1], pad], axis=1)

  p1 = _sc_aggregate(N1, E1 // NW)(xpad, edge_index1)
  hpad = _tc_layer1(p1, x[:N2], W1l, W1r, b1)
  p2 = _sc_aggregate(NSEG, E2 // NW)(hpad, edge_index2)
  return _tc_layer2(p2, hpad, W2l, W2r, b2)


# speedup vs baseline: 2.1156x; 2.1156x over previous
import functools

import jax
import jax.numpy as jnp
from jax import lax
from jax.experimental import pallas as pl
from jax.experimental.pallas import tpu as pltpu
from jax.experimental.pallas import tpu_sc as plsc

N0, N1, N2 = 10000, 4000, 1000
E1, E2 = 320000, 64000
D = 128
DP = 136
NC, NS = 2, 16
NW = NC * NS
NSEG = 1024


def _sc_aggregate(n_rows, epw, NB, CH):
  mesh = plsc.VectorSubcoreMesh(
      core_axis_name="c", subcore_axis_name="s", num_cores=NC,
      num_subcores=NS)
  zrows = NSEG // NS
  trows = n_rows // NS
  qcap = epw + NB * CH

  @functools.partial(
      pl.kernel,
      out_type=jax.ShapeDtypeStruct((NC, NSEG, DP), jnp.float32),
      mesh=mesh,
      scratch_types=[
          pltpu.VMEM((epw,), jnp.int32),
          pltpu.VMEM((epw,), jnp.int32),
          pltpu.VMEM((qcap,), jnp.int32),
          pltpu.VMEM((qcap,), jnp.int32),
          pltpu.VMEM((NB, CH), jnp.int32),
          pltpu.VMEM((NB, CH), jnp.int32),
          pltpu.VMEM((NB, CH, DP), jnp.float32),
          pltpu.VMEM_SHARED((n_rows, DP), jnp.float32),
          pltpu.VMEM_SHARED((NSEG, DP), jnp.float32),
          pltpu.SemaphoreType.DMA,
          pltpu.SemaphoreType.DMA((NB,)),
          pltpu.SemaphoreType.DMA((NB,)),
      ],
      compiler_params=pltpu.CompilerParams(use_tc_tiling_on_sc=False,
                                           needs_layout_passes=False),
  )
  def agg(table_hbm, edges_hbm, out_hbm, src_v, dst_v, qs, qd, qs2, qd2,
          rows, tbl, acc, semt, semg, sems):
    cid = lax.axis_index("c")
    sid = lax.axis_index("s")
    wid = cid * NS + sid

    tdma = pltpu.make_async_copy(
        table_hbm.at[pl.ds(sid * trows, trows)],
        tbl.at[pl.ds(sid * trows, trows)], semt)
    tdma.start()

    zero = jnp.zeros((16,), jnp.float32)
    for r in range(zrows):
      for k in range(D // 16):
        rows[0, r, pl.ds(k * 16, 16)] = zero
      rows[0, r, pl.ds(DP - 16, 16)] = zero
    pltpu.sync_copy(rows.at[0, pl.ds(0, zrows)],
                    acc.at[pl.ds(sid * zrows, zrows)])

    pltpu.sync_copy(edges_hbm.at[0, pl.ds(wid * epw, epw)], src_v)
    pltpu.sync_copy(edges_hbm.at[1, pl.ds(wid * epw, epw)], dst_v)

    iota = lax.iota(jnp.int32, 16)

    def scan_body(i, qn):
      off = pl.multiple_of(i * 16, 16)
      s = src_v[pl.ds(off, 16)]
      d = dst_v[pl.ds(off, 16)]
      m = d < NSEG
      pos = qn + plsc.cumsum(m.astype(jnp.int32)) - 1
      plsc.store_scatter(qs, [pos], s, mask=m)
      plsc.store_scatter(qd, [pos], d, mask=m)
      return qn + plsc.all_reduce_population_count(m)

    qn_v = lax.fori_loop(0, epw // 16, scan_body, jnp.zeros((16,), jnp.int32))
    for k in range(((NB - 1) * CH) // 16):
      pos = qn_v + k * 16 + iota
      plsc.store_scatter(qs, [pos], jnp.zeros((16,), jnp.int32))
      plsc.store_scatter(qd, [pos], jnp.full((16,), NSEG - 8, jnp.int32))
    qn = jnp.max(qn_v)
    nch = jnp.maximum(qn // CH + 1, NB - 1)
    tdma.wait()
    plsc.subcore_barrier()

    def stage(c, b):
      for k in range(CH // 16):
        pos = c * CH + k * 16 + iota
        qs2[b, pl.ds(k * 16, 16)] = plsc.load_gather(qs, [pos])
        qd2[b, pl.ds(k * 16, 16)] = plsc.load_gather(qd, [pos])

    def gather(b):
      return pltpu.make_async_copy(
          tbl.at[qs2.at[b]], rows.at[b], semg.at[b])

    def scatter_start(b):
      pltpu.async_copy(rows.at[b], acc.at[qd2.at[b]], sems.at[b], add=True)

    def scatter_wait(b):
      pltpu.make_async_copy(rows.at[b], acc.at[qd2.at[b]], sems.at[b]).wait()

    for b in range(NB - 1):
      stage(b, b)
      gather(b).start()

    def body(c, carry):
      b = c % NB
      ob = (c + NB - 1) % NB
      gather(b).wait()
      scatter_start(b)

      @pl.when(c >= 1)
      def _drain():
        scatter_wait(ob)

      stage(c + NB - 1, ob)
      gather(ob).start()
      return carry

    lax.fori_loop(0, nch - (NB - 1), body, 0)

    def tail(k, carry):
      c = nch - (NB - 1) + k
      b = c % NB
      gather(b).wait()
      pltpu.sync_copy(rows.at[b], acc.at[qd2.at[b]], add=True)
      return carry

    lax.fori_loop(0, NB - 1, tail, 0)

    @pl.when(nch > NB - 1)
    def _drain_last_async():
      scatter_wait((nch - NB) % NB)

    plsc.subcore_barrier()
    pltpu.sync_copy(acc.at[pl.ds(sid * zrows, zrows)],
                    out_hbm.at[cid, pl.ds(sid * zrows, zrows)])

  return agg


def _tc_layer1(p, x, W1l, W1r, b1):
  def body(p_ref, x_ref, wl_ref, wr_ref, b_ref, out_ref):
    ps = p_ref[0, :N2] + p_ref[1, :N2]
    s = ps[:, :D]
    cnt = ps[:, D:D + 1]
    mean = s / jnp.maximum(cnt, 1.0)
    h = mean @ wl_ref[...] + x_ref[...] @ wr_ref[...] + b_ref[...]
    h = jnp.maximum(h, 0.0)
    hp = jnp.concatenate(
        [h, jnp.ones((N2, 1), jnp.float32),
         jnp.zeros((N2, DP - D - 1), jnp.float32)], axis=1)
    out_ref[...] = jnp.concatenate(
        [hp, jnp.zeros((NSEG - N2, DP), jnp.float32)], axis=0)

  return pl.pallas_call(
      body,
      out_shape=jax.ShapeDtypeStruct((NSEG, DP), jnp.float32),
  )(p, x, W1l, W1r, b1.reshape(1, D))


def _tc_layer2(p, hpad, W2l, W2r, b2):
  def body(p_ref, h_ref, wl_ref, wr_ref, b_ref, out_ref):
    ps = p_ref[0, :N2] + p_ref[1, :N2]
    s = ps[:, :D]
    cnt = ps[:, D:D + 1]
    mean = s / jnp.maximum(cnt, 1.0)
    z = mean @ wl_ref[...] + h_ref[:N2, :D] @ wr_ref[...] + b_ref[...]
    m = jnp.max(z, axis=-1, keepdims=True)
    e = jnp.exp(z - m)
    lse = jnp.log(jnp.sum(e, axis=-1, keepdims=True)) + m
    out_ref[...] = z - lse

  return pl.pallas_call(
      body,
      out_shape=jax.ShapeDtypeStruct((N2, D), jnp.float32),
  )(p, hpad, W2l, W2r, b2.reshape(1, D))


def kernel(x, edge_index1, edge_index2, W1l, W1r, b1, W2l, W2r, b2):
  pad = jnp.concatenate(
      [jnp.ones((N1, 1), jnp.float32),
       jnp.zeros((N1, DP - D - 1), jnp.float32)], axis=1)
  xpad = jnp.concatenate([x[:N1], pad], axis=1)

  p1 = _sc_aggregate(N1, E1 // NW, 3, 112)(xpad, edge_index1)
  hpad = _tc_layer1(p1, x[:N2], W1l, W1r, b1)
  p2 = _sc_aggregate(NSEG, E2 // NW, 4, 128)(hpad, edge_index2)
  return _tc_layer2(p2, hpad, W2l, W2r, b2)


# speedup vs baseline: 2.1911x; 1.0357x over previous
import functools

import jax
import jax.numpy as jnp
from jax import lax
from jax.experimental import pallas as pl
from jax.experimental.pallas import tpu as pltpu
from jax.experimental.pallas import tpu_sc as plsc

N0, N1, N2 = 10000, 4000, 1000
E1, E2 = 320000, 64000
D = 128
CW = 16
NC, NS = 2, 16
NW = NC * NS
NSEG = 1024


def _sc_aggregate(n_rows, epw, NB, CH):
  mesh = plsc.VectorSubcoreMesh(
      core_axis_name="c", subcore_axis_name="s", num_cores=NC,
      num_subcores=NS)
  zrows = NSEG // NS
  trows = n_rows // NS
  qcap = epw + NB * CH

  @functools.partial(
      pl.kernel,
      out_type=(jax.ShapeDtypeStruct((NC, NSEG, D), jnp.float32),
                jax.ShapeDtypeStruct((NC, NSEG, CW), jnp.float32)),
      mesh=mesh,
      scratch_types=[
          pltpu.VMEM((epw,), jnp.int32),
          pltpu.VMEM((epw,), jnp.int32),
          pltpu.VMEM((qcap,), jnp.int32),
          pltpu.VMEM((qcap,), jnp.int32),
          pltpu.VMEM((NB, CH), jnp.int32),
          pltpu.VMEM((NB, CH), jnp.int32),
          pltpu.VMEM((NB, CH, D), jnp.float32),
          pltpu.VMEM((CH, CW), jnp.float32),
          pltpu.VMEM_SHARED((n_rows, D), jnp.float32),
          pltpu.VMEM_SHARED((NSEG, D), jnp.float32),
          pltpu.VMEM_SHARED((NSEG, CW), jnp.float32),
          pltpu.SemaphoreType.DMA,
          pltpu.SemaphoreType.DMA((NB,)),
          pltpu.SemaphoreType.DMA((NB,)),
          pltpu.SemaphoreType.DMA((NB,)),
      ],
      compiler_params=pltpu.CompilerParams(use_tc_tiling_on_sc=False,
                                           needs_layout_passes=False),
  )
  def agg(table_hbm, edges_hbm, outd_hbm, outc_hbm, src_v, dst_v, qs, qd,
          qs2, qd2, rows, cbuf, tbl, accD, accC, semt, semg, sems, semc):
    cid = lax.axis_index("c")
    sid = lax.axis_index("s")
    wid = cid * NS + sid

    tdma = pltpu.make_async_copy(
        table_hbm.at[pl.ds(sid * trows, trows)],
        tbl.at[pl.ds(sid * trows, trows)], semt)
    tdma.start()

    zero = jnp.zeros((16,), jnp.float32)
    for r in range(zrows):
      for k in range(D // 16):
        rows[0, r, pl.ds(k * 16, 16)] = zero
    pltpu.sync_copy(rows.at[0, pl.ds(0, zrows)],
                    accD.at[pl.ds(sid * zrows, zrows)])
    for r in range(CH):
      cbuf[r, pl.ds(0, 16)] = zero
    pltpu.sync_copy(cbuf.at[pl.ds(0, zrows)],
                    accC.at[pl.ds(sid * zrows, zrows)])
    iota = lax.iota(jnp.int32, 16)
    cvec = jnp.where(iota == 0, 1.0, 0.0).astype(jnp.float32)
    for r in range(CH):
      cbuf[r, pl.ds(0, 16)] = cvec

    pltpu.sync_copy(edges_hbm.at[0, pl.ds(wid * epw, epw)], src_v)
    pltpu.sync_copy(edges_hbm.at[1, pl.ds(wid * epw, epw)], dst_v)

    def scan_body(i, qn):
      off = pl.multiple_of(i * 16, 16)
      s = src_v[pl.ds(off, 16)]
      d = dst_v[pl.ds(off, 16)]
      m = d < NSEG
      pos = qn + plsc.cumsum(m.astype(jnp.int32)) - 1
      plsc.store_scatter(qs, [pos], s, mask=m)
      plsc.store_scatter(qd, [pos], d, mask=m)
      return qn + plsc.all_reduce_population_count(m)

    qn_v = lax.fori_loop(0, epw // 16, scan_body, jnp.zeros((16,), jnp.int32))
    for k in range(((NB - 1) * CH) // 16):
      pos = qn_v + k * 16 + iota
      plsc.store_scatter(qs, [pos], jnp.zeros((16,), jnp.int32))
      plsc.store_scatter(qd, [pos], jnp.full((16,), NSEG - 8, jnp.int32))
    qn = jnp.max(qn_v)
    nch = jnp.maximum(qn // CH + 1, NB - 1)
    tdma.wait()
    plsc.subcore_barrier()

    def stage(c, b):
      for k in range(CH // 16):
        pos = c * CH + k * 16 + iota
        qs2[b, pl.ds(k * 16, 16)] = plsc.load_gather(qs, [pos])
        qd2[b, pl.ds(k * 16, 16)] = plsc.load_gather(qd, [pos])

    def gather(b):
      return pltpu.make_async_copy(
          tbl.at[qs2.at[b]], rows.at[b], semg.at[b])

    def scatter_start(b):
      pltpu.async_copy(rows.at[b], accD.at[qd2.at[b]], sems.at[b], add=True)
      pltpu.async_copy(cbuf, accC.at[qd2.at[b]], semc.at[b], add=True)

    def scatter_wait(b):
      pltpu.make_async_copy(rows.at[b], accD.at[qd2.at[b]], sems.at[b]).wait()
      pltpu.make_async_copy(cbuf, accC.at[qd2.at[b]], semc.at[b]).wait()

    for b in range(NB - 1):
      stage(b, b)
      gather(b).start()

    def body(c, carry):
      b = c % NB
      ob = (c + NB - 1) % NB
      gather(b).wait()
      scatter_start(b)

      @pl.when(c >= 1)
      def _drain():
        scatter_wait(ob)

      stage(c + NB - 1, ob)
      gather(ob).start()
      return carry

    lax.fori_loop(0, nch - (NB - 1), body, 0)

    def tail(k, carry):
      c = nch - (NB - 1) + k
      b = c % NB
      gather(b).wait()
      pltpu.sync_copy(rows.at[b], accD.at[qd2.at[b]], add=True)
      pltpu.sync_copy(cbuf, accC.at[qd2.at[b]], add=True)
      return carry

    lax.fori_loop(0, NB - 1, tail, 0)

    @pl.when(nch > NB - 1)
    def _drain_last_async():
      scatter_wait((nch - NB) % NB)

    plsc.subcore_barrier()
    pltpu.sync_copy(accD.at[pl.ds(sid * zrows, zrows)],
                    outd_hbm.at[cid, pl.ds(sid * zrows, zrows)])
    pltpu.sync_copy(accC.at[pl.ds(sid * zrows, zrows)],
                    outc_hbm.at[cid, pl.ds(sid * zrows, zrows)])

  return agg


def _tc_layer1(pd, pc, x, W1l, W1r, b1):
  def body(pd_ref, pc_ref, x_ref, wl_ref, wr_ref, b_ref, out_ref):
    s = pd_ref[0, :N2] + pd_ref[1, :N2]
    cnt = pc_ref[0, :N2, 0:1] + pc_ref[1, :N2, 0:1]
    mean = s / jnp.maximum(cnt, 1.0)
    h = mean @ wl_ref[...] + x_ref[...] @ wr_ref[...] + b_ref[...]
    h = jnp.maximum(h, 0.0)
    out_ref[...] = jnp.concatenate(
        [h, jnp.zeros((NSEG - N2, D), jnp.float32)], axis=0)

  return pl.pallas_call(
      body,
      grid=(1,),
      in_specs=[
          pl.BlockSpec((NC, NSEG, D), lambda i: (0, 0, 0)),
          pl.BlockSpec((NC, NSEG, CW), lambda i: (0, 0, 0)),
          pl.BlockSpec((N2, D), lambda i: (0, 0)),
          pl.BlockSpec((D, D), lambda i: (0, 0)),
          pl.BlockSpec((D, D), lambda i: (0, 0)),
          pl.BlockSpec((1, D), lambda i: (0, 0)),
      ],
      out_specs=pl.BlockSpec((NSEG, D), lambda i: (0, 0)),
      out_shape=jax.ShapeDtypeStruct((NSEG, D), jnp.float32),
  )(pd, pc, x, W1l, W1r, b1.reshape(1, D))


def _tc_layer2(pd, pc, h, W2l, W2r, b2):
  def body(pd_ref, pc_ref, h_ref, wl_ref, wr_ref, b_ref, out_ref):
    s = pd_ref[0, :N2] + pd_ref[1, :N2]
    cnt = pc_ref[0, :N2, 0:1] + pc_ref[1, :N2, 0:1]
    mean = s / jnp.maximum(cnt, 1.0)
    z = mean @ wl_ref[...] + h_ref[:N2] @ wr_ref[...] + b_ref[...]
    m = jnp.max(z, axis=-1, keepdims=True)
    e = jnp.exp(z - m)
    lse = jnp.log(jnp.sum(e, axis=-1, keepdims=True)) + m
    out_ref[...] = z - lse

  return pl.pallas_call(
      body,
      out_shape=jax.ShapeDtypeStruct((N2, D), jnp.float32),
  )(pd, pc, h, W2l, W2r, b2.reshape(1, D))


def kernel(x, edge_index1, edge_index2, W1l, W1r, b1, W2l, W2r, b2):
  p1d, p1c = _sc_aggregate(N1, E1 // NW, 3, 112)(x[:N1], edge_index1)
  h = _tc_layer1(p1d, p1c, x, W1l, W1r, b1)
  p2d, p2c = _sc_aggregate(NSEG, E2 // NW, 4, 128)(h, edge_index2)
  return _tc_layer2(p2d, p2c, h, W2l, W2r, b2)
